# Initial kernel scaffold; baseline (speedup 1.0000x reference)
import jax
import jax.numpy as jnp
from jax.experimental import pallas as pl


def kernel(x, edge_index, edge_attr, W_in, b_in, lin_W, lin_b, att_W, att_b, ln_g, ln_b, p1_W, p1_b, p2_W, p2_b):
    raise NotImplementedError("write your pallas kernel here")


# speedup vs baseline: 3.7311x; 3.7311x over previous
import functools

import jax
import jax.numpy as jnp
from jax import lax
from jax.experimental import pallas as pl
from jax.experimental.pallas import tpu as pltpu
from jax.experimental.pallas import tpu_sc as plsc

N = 10000
E = 160000
D = 128
L = 3

NWORKERS = 32
CH = 128
EPAD = 163840
EPW = EPAD // NWORKERS
NCHUNK = EPW // CH
NPAD = 10240
ROWS_PW = NPAD // 16

BN = 1000
GRID = N // BN


def _sc_edges_body(src_hbm, dst_hbm, ea_hbm, hw_hbm, asrc_hbm, adst_hbm, out_hbm,
              src_v, dst_v, ea_v, cf_v, rows_v, asrc_v, adst_v, acc, sem):
    cid = lax.axis_index("c")
    sid = lax.axis_index("s")
    wid = cid * 16 + sid

    def _zrow(r, carry):
        for j in range(8):
            rows_v[r, pl.ds(j * 16, 16)] = jnp.zeros((16,), jnp.float32)
        return carry
    lax.fori_loop(0, CH, _zrow, 0)

    row0 = sid * ROWS_PW
    for k in range(ROWS_PW // CH):
        pltpu.sync_copy(rows_v, acc.at[pl.ds(row0 + k * CH, CH)])

    pltpu.sync_copy(asrc_hbm, asrc_v)
    pltpu.sync_copy(adst_hbm, adst_v)
    nrow0 = wid * NCHUNK
    pltpu.sync_copy(src_hbm.at[pl.ds(nrow0, NCHUNK)], src_v)
    pltpu.sync_copy(dst_hbm.at[pl.ds(nrow0, NCHUNK)], dst_v)

    plsc.subcore_barrier()

    def _chunk(t, carry):
        pltpu.sync_copy(ea_hbm.at[nrow0 + t], ea_v)
        pltpu.async_copy(hw_hbm.at[src_v.at[t]], rows_v, sem).wait()
        for g in range(CH // 16):
            si = src_v[t, pl.ds(g * 16, 16)]
            di = dst_v[t, pl.ds(g * 16, 16)]
            z = plsc.load_gather(asrc_v, [si]) + plsc.load_gather(adst_v, [di])
            att = 1.0 / (1.0 + jnp.exp(-z))
            cf_v[pl.ds(g * 16, 16)] = ea_v[pl.ds(g * 16, 16)] * att

        def _scale(e, c2):
            cfs = plsc.load_gather(cf_v, [jnp.full((16,), e, jnp.int32)])
            for j in range(8):
                rows_v[e, pl.ds(j * 16, 16)] = rows_v[e, pl.ds(j * 16, 16)] * cfs
            return c2
        lax.fori_loop(0, CH, _scale, 0)

        pltpu.sync_copy(rows_v, acc.at[dst_v.at[t]], add=True)
        return carry
    lax.fori_loop(0, NCHUNK, _chunk, 0)

    plsc.subcore_barrier()

    for k in range(ROWS_PW // CH):
        pltpu.sync_copy(acc.at[pl.ds(row0 + k * CH, CH)], rows_v)
        pltpu.sync_copy(rows_v, out_hbm.at[cid, pl.ds(row0 + k * CH, CH)])


@functools.lru_cache(maxsize=1)
def _get_sc_edges():
    mesh = plsc.VectorSubcoreMesh(core_axis_name="c", subcore_axis_name="s")
    return pl.kernel(
        _sc_edges_body,
        mesh=mesh,
        compiler_params=pltpu.CompilerParams(needs_layout_passes=False),
        out_type=jax.ShapeDtypeStruct((2, NPAD, D), jnp.float32),
        scratch_types=[
            pltpu.VMEM((NCHUNK, CH), jnp.int32),
            pltpu.VMEM((NCHUNK, CH), jnp.int32),
            pltpu.VMEM((CH,), jnp.float32),
            pltpu.VMEM((CH,), jnp.float32),
            pltpu.VMEM((CH, D), jnp.float32),
            pltpu.VMEM((N,), jnp.float32),
            pltpu.VMEM((N,), jnp.float32),
            pltpu.VMEM_SHARED((NPAD, D), jnp.float32),
            pltpu.SemaphoreType.DMA,
        ],
    )


def _tc_in_body(x_ref, win_ref, bin_ref, wl_ref, bl_ref, wa_ref, ba_ref,
                h_ref, hw_ref, az_ref):
    h = jnp.dot(x_ref[...], win_ref[...], preferred_element_type=jnp.float32) + bin_ref[...]
    h_ref[...] = h
    hw_ref[...] = jnp.dot(h, wl_ref[...], preferred_element_type=jnp.float32) + bl_ref[...]
    az_ref[...] = jnp.dot(h, wa_ref[...], preferred_element_type=jnp.float32) + ba_ref[...]


def _ln_relu_res(h, hw, p0, p1, g, b):
    h2 = p0 + p1 + hw
    mu = jnp.mean(h2, axis=-1, keepdims=True)
    xc = h2 - mu
    var = jnp.mean(xc * xc, axis=-1, keepdims=True)
    y = xc * lax.rsqrt(var + 1e-5) * g + b
    return jnp.maximum(y, 0.0) + h


def _tc_mid_body(h_ref, hw_ref, p0_ref, p1_ref, g_ref, b_ref,
                 wl_ref, bl_ref, wa_ref, ba_ref,
                 hn_ref, hwn_ref, azn_ref):
    hn = _ln_relu_res(h_ref[...], hw_ref[...], p0_ref[0], p1_ref[0],
                      g_ref[...], b_ref[...])
    hn_ref[...] = hn
    hwn_ref[...] = jnp.dot(hn, wl_ref[...], preferred_element_type=jnp.float32) + bl_ref[...]
    azn_ref[...] = jnp.dot(hn, wa_ref[...], preferred_element_type=jnp.float32) + ba_ref[...]


def _tc_fin_body(h_ref, hw_ref, p0_ref, p1_ref, g_ref, b_ref,
                 p1w_ref, p1b_ref, p2w_ref, p2b_ref,
                 hn_ref, gout_ref, pout_ref, acc_ref):
    i = pl.program_id(0)
    hn = _ln_relu_res(h_ref[...], hw_ref[...], p0_ref[0], p1_ref[0],
                      g_ref[...], b_ref[...])
    hn_ref[...] = hn
    part = jnp.sum(hn, axis=0, keepdims=True)

    @pl.when(i == 0)
    def _():
        acc_ref[...] = part

    @pl.when(i > 0)
    def _():
        acc_ref[...] = acc_ref[...] + part

    @pl.when(i == GRID - 1)
    def _():
        g = acc_ref[...] * (1.0 / N)
        gout_ref[...] = g
        t = jnp.maximum(
            jnp.dot(g, p1w_ref[...], preferred_element_type=jnp.float32) + p1b_ref[...], 0.0)
        z = jnp.dot(t, p2w_ref[...], preferred_element_type=jnp.float32) + p2b_ref[...]
        pout_ref[...] = 1.0 / (1.0 + jnp.exp(-z))


_ROW = lambda i: (i, 0)
_FIX = lambda i: (0, 0)
_B_ROW = pl.BlockSpec((BN, D), _ROW)
_B_W = pl.BlockSpec((D, D), _FIX)
_B_B = pl.BlockSpec((1, D), _FIX)
_B_P0 = pl.BlockSpec((1, BN, D), lambda i: (0, i, 0))
_B_P1 = pl.BlockSpec((1, BN, D), lambda i: (1, i, 0))
_SDS = jax.ShapeDtypeStruct

_tc_in = pl.pallas_call(
    _tc_in_body,
    grid=(GRID,),
    in_specs=[_B_ROW, _B_W, _B_B, _B_W, _B_B, _B_W, _B_B],
    out_specs=[_B_ROW, _B_ROW, _B_ROW],
    out_shape=[_SDS((N, D), jnp.float32)] * 3,
)

_tc_mid = pl.pallas_call(
    _tc_mid_body,
    grid=(GRID,),
    in_specs=[_B_ROW, _B_ROW, _B_P0, _B_P1, _B_B, _B_B, _B_W, _B_B, _B_W, _B_B],
    out_specs=[_B_ROW, _B_ROW, _B_ROW],
    out_shape=[_SDS((N, D), jnp.float32)] * 3,
)

_tc_fin = pl.pallas_call(
    _tc_fin_body,
    grid=(GRID,),
    in_specs=[_B_ROW, _B_ROW, _B_P0, _B_P1, _B_B, _B_B, _B_W, _B_B, _B_W, _B_B],
    out_specs=[_B_ROW, _B_B, _B_B],
    out_shape=[_SDS((N, D), jnp.float32), _SDS((1, D), jnp.float32), _SDS((1, D), jnp.float32)],
    scratch_shapes=[pltpu.VMEM((1, D), jnp.float32)],
)


def kernel(x, edge_index, edge_attr, W_in, b_in, lin_W, lin_b, att_W, att_b,
           ln_g, ln_b, p1_W, p1_b, p2_W, p2_b):
    f32 = jnp.float32
    pad = EPAD - E
    src2d = jnp.concatenate([edge_index[0], jnp.zeros((pad,), jnp.int32)]).reshape(EPAD // CH, CH)
    dst2d = jnp.concatenate([edge_index[1], jnp.zeros((pad,), jnp.int32)]).reshape(EPAD // CH, CH)
    ea2d = jnp.concatenate([edge_attr, jnp.zeros((pad,), f32)]).reshape(EPAD // CH, CH)

    wa = jnp.concatenate(
        [att_W[:, :D], att_W[:, D:], jnp.zeros((L, D, D - 2), f32)], axis=2)
    ba = jnp.zeros((L, 1, D), f32).at[:, 0, 0].set(att_b[:, 0])

    H = p1_W.shape[1]
    p1w = jnp.zeros((D, D), f32).at[:, :H].set(p1_W)
    p1b = jnp.zeros((1, D), f32).at[0, :H].set(p1_b)
    p2w = jnp.zeros((D, D), f32).at[:H, :1].set(p2_W)
    p2b = jnp.zeros((1, D), f32).at[0, 0].set(p2_b[0])

    h, hw, az = _tc_in(x, W_in, b_in.reshape(1, D), lin_W[0], lin_b[0].reshape(1, D),
                       wa[0], ba[0])
    for i in range(L):
        adst = az[:, 0]
        asrc = az[:, 1]
        parts = _get_sc_edges()(src2d, dst2d, ea2d, hw, asrc, adst)
        lg = ln_g[i].reshape(1, D)
        lb = ln_b[i].reshape(1, D)
        if i < L - 1:
            h, hw, az = _tc_mid(h, hw, parts, parts, lg, lb,
                                lin_W[i + 1], lin_b[i + 1].reshape(1, D),
                                wa[i + 1], ba[i + 1])
        else:
            h, g, ppad = _tc_fin(h, hw, parts, parts, lg, lb, p1w, p1b, p2w, p2b)
    prob = ppad[:, :1]
    return (prob, h, g)
